# baseline (speedup 1.0000x reference)
import functools

import jax
import jax.numpy as jnp
from jax import lax
from jax.experimental import pallas as pl
from jax.experimental.pallas import tpu as pltpu
from jax.experimental.pallas import tpu_sc as plsc

HIDDEN = 1024
NUM_EXPERTS = 8
TOP_K = 2
BLOCK = 2048

N_TOKENS = 32768
SC_NW = 32
SC_TPW = N_TOKENS // SC_NW
SC_CT = 512
SC_NCH = SC_TPW // SC_CT


def _scores_block(x_ref, wt_ref, scores_ref, st_ref):
    xb = x_ref[...]
    wt = wt_ref[...]
    logits = jnp.dot(xb, wt, preferred_element_type=jnp.float32)
    sp = jnp.maximum(logits, 0.0) + jnp.log(1.0 + jnp.exp(-jnp.abs(logits)))
    norm = jnp.sum(sp, axis=1, keepdims=True)
    sn = sp / jnp.maximum(norm, 1e-12)
    scores_ref[...] = sn
    st_ref[...] = sn.T


def _sc_topk_body(st_hbm, o_hbm, sbuf, obuf):
    wid = lax.axis_index("s") * 2 + lax.axis_index("c")
    base = wid * SC_TPW
    lane = lax.broadcasted_iota(jnp.int32, (16,), 0)

    def group(g, carry):
        v = [sbuf[e, pl.ds(g * 16, 16)] for e in range(NUM_EXPERTS)]

        m1 = v[0]
        for e in range(1, NUM_EXPERTS):
            m1 = jnp.maximum(m1, v[e])
        i1 = jnp.full((16,), NUM_EXPERTS - 1, jnp.int32)
        for e in range(NUM_EXPERTS - 2, -1, -1):
            i1 = jnp.where(v[e] == m1, jnp.full((16,), e, jnp.int32), i1)

        mv = [jnp.where(i1 == e, -1.0, v[e]) for e in range(NUM_EXPERTS)]
        m2 = mv[0]
        for e in range(1, NUM_EXPERTS):
            m2 = jnp.maximum(m2, mv[e])
        i2 = jnp.full((16,), NUM_EXPERTS - 1, jnp.int32)
        for e in range(NUM_EXPERTS - 2, -1, -1):
            i2 = jnp.where(mv[e] == m2, jnp.full((16,), e, jnp.int32), i2)

        obase = g * 64 + lane * 4
        plsc.store_scatter(obuf, [obase], plsc.bitcast(m1, jnp.int32))
        plsc.store_scatter(obuf, [obase + 1], plsc.bitcast(m2, jnp.int32))
        plsc.store_scatter(obuf, [obase + 2], i1)
        plsc.store_scatter(obuf, [obase + 3], i2)
        return carry

    for ch in range(SC_NCH):
        t0 = base + ch * SC_CT
        pltpu.sync_copy(st_hbm.at[:, pl.ds(t0, SC_CT)], sbuf)
        lax.fori_loop(0, SC_CT // 16, group, 0)
        pltpu.sync_copy(obuf, o_hbm.at[pl.ds(t0 * 4, SC_CT * 4)])


@functools.partial(
    pl.kernel,
    mesh=plsc.VectorSubcoreMesh(core_axis_name="c", subcore_axis_name="s"),
    compiler_params=pltpu.CompilerParams(needs_layout_passes=False),
    out_type=jax.ShapeDtypeStruct((N_TOKENS * 4,), jnp.int32),
    scratch_types=[
        pltpu.MemorySpace.VMEM((NUM_EXPERTS, SC_CT), jnp.float32),
        pltpu.MemorySpace.VMEM((SC_CT * 4,), jnp.int32),
    ],
)
def _sc_topk(st_hbm, o_hbm, sbuf, obuf):
    _sc_topk_body(st_hbm, o_hbm, sbuf, obuf)


@jax.jit
def _router(x2d, wt):
    n = x2d.shape[0]
    grid = n // BLOCK
    scores, scores_t = pl.pallas_call(
        _scores_block,
        grid=(grid,),
        in_specs=[
            pl.BlockSpec((BLOCK, HIDDEN), lambda i: (i, 0)),
            pl.BlockSpec((HIDDEN, NUM_EXPERTS), lambda i: (0, 0)),
        ],
        out_specs=[
            pl.BlockSpec((BLOCK, NUM_EXPERTS), lambda i: (i, 0)),
            pl.BlockSpec((NUM_EXPERTS, BLOCK), lambda i: (0, i)),
        ],
        out_shape=[
            jax.ShapeDtypeStruct((n, NUM_EXPERTS), jnp.float32),
            jax.ShapeDtypeStruct((NUM_EXPERTS, n), jnp.float32),
        ],
    )(x2d, wt)
    o_flat = _sc_topk(scores_t).reshape(n, 4)
    weights = jax.lax.bitcast_convert_type(o_flat[:, :TOP_K], jnp.float32)
    return scores, weights, o_flat[:, TOP_K:]


def kernel(x, W):
    x2d = x.reshape(-1, x.shape[-1])
    scores, weights, indices = _router(x2d, W.T)
    return scores, weights, indices, jnp.float32(0.0)


# speedup vs baseline: 1.7617x; 1.7617x over previous
import functools

import jax
import jax.numpy as jnp
from jax import lax
from jax.experimental import pallas as pl
from jax.experimental.pallas import tpu as pltpu
from jax.experimental.pallas import tpu_sc as plsc

HIDDEN = 1024
NUM_EXPERTS = 8
TOP_K = 2
BLOCK = 2048

N_TOKENS = 32768
SC_NW = 32
SC_TPW = N_TOKENS // SC_NW
SC_CT = 512
SC_NCH = SC_TPW // SC_CT


def _scores_block(x_ref, wt_ref, scores_ref, st_ref):
    xb = x_ref[...]
    wt = wt_ref[...]
    logits = jnp.dot(xb, wt, preferred_element_type=jnp.float32)
    sp = jnp.maximum(logits, 0.0) + jnp.log(1.0 + jnp.exp(-jnp.abs(logits)))
    norm = jnp.sum(sp, axis=1, keepdims=True)
    sn = sp / jnp.maximum(norm, 1e-12)
    scores_ref[...] = sn
    st_ref[...] = sn.T


def _sc_topk_body(st_hbm, w1_hbm, w2_hbm, i1_hbm, i2_hbm,
                  sbuf, w1b, w2b, i1b, i2b):
    wid = lax.axis_index("s") * 2 + lax.axis_index("c")
    base = wid * SC_TPW

    def group(g, carry):
        v = [sbuf[e, pl.ds(g * 16, 16)] for e in range(NUM_EXPERTS)]

        m1 = v[0]
        for e in range(1, NUM_EXPERTS):
            m1 = jnp.maximum(m1, v[e])
        i1 = jnp.full((16,), NUM_EXPERTS - 1, jnp.int32)
        for e in range(NUM_EXPERTS - 2, -1, -1):
            i1 = jnp.where(v[e] == m1, jnp.full((16,), e, jnp.int32), i1)

        mv = [jnp.where(i1 == e, -1.0, v[e]) for e in range(NUM_EXPERTS)]
        m2 = mv[0]
        for e in range(1, NUM_EXPERTS):
            m2 = jnp.maximum(m2, mv[e])
        i2 = jnp.full((16,), NUM_EXPERTS - 1, jnp.int32)
        for e in range(NUM_EXPERTS - 2, -1, -1):
            i2 = jnp.where(mv[e] == m2, jnp.full((16,), e, jnp.int32), i2)

        sl = pl.ds(g * 16, 16)
        w1b[sl] = m1
        w2b[sl] = m2
        i1b[sl] = i1
        i2b[sl] = i2
        return carry

    for ch in range(SC_NCH):
        t0 = base + ch * SC_CT
        pltpu.sync_copy(st_hbm.at[:, pl.ds(t0, SC_CT)], sbuf)
        lax.fori_loop(0, SC_CT // 16, group, 0)
        sl = pl.ds(t0, SC_CT)
        pltpu.sync_copy(w1b, w1_hbm.at[sl])
        pltpu.sync_copy(w2b, w2_hbm.at[sl])
        pltpu.sync_copy(i1b, i1_hbm.at[sl])
        pltpu.sync_copy(i2b, i2_hbm.at[sl])


@functools.partial(
    pl.kernel,
    mesh=plsc.VectorSubcoreMesh(core_axis_name="c", subcore_axis_name="s"),
    out_type=[
        jax.ShapeDtypeStruct((N_TOKENS,), jnp.float32),
        jax.ShapeDtypeStruct((N_TOKENS,), jnp.float32),
        jax.ShapeDtypeStruct((N_TOKENS,), jnp.int32),
        jax.ShapeDtypeStruct((N_TOKENS,), jnp.int32),
    ],
    scratch_types=[
        pltpu.MemorySpace.VMEM((NUM_EXPERTS, SC_CT), jnp.float32),
        pltpu.MemorySpace.VMEM((SC_CT,), jnp.float32),
        pltpu.MemorySpace.VMEM((SC_CT,), jnp.float32),
        pltpu.MemorySpace.VMEM((SC_CT,), jnp.int32),
        pltpu.MemorySpace.VMEM((SC_CT,), jnp.int32),
    ],
)
def _sc_topk(st_hbm, w1_hbm, w2_hbm, i1_hbm, i2_hbm, sbuf, w1b, w2b, i1b, i2b):
    _sc_topk_body(st_hbm, w1_hbm, w2_hbm, i1_hbm, i2_hbm,
                  sbuf, w1b, w2b, i1b, i2b)


@jax.jit
def _router(x2d, wt):
    n = x2d.shape[0]
    grid = n // BLOCK
    scores, scores_t = pl.pallas_call(
        _scores_block,
        grid=(grid,),
        in_specs=[
            pl.BlockSpec((BLOCK, HIDDEN), lambda i: (i, 0)),
            pl.BlockSpec((HIDDEN, NUM_EXPERTS), lambda i: (0, 0)),
        ],
        out_specs=[
            pl.BlockSpec((BLOCK, NUM_EXPERTS), lambda i: (i, 0)),
            pl.BlockSpec((NUM_EXPERTS, BLOCK), lambda i: (0, i)),
        ],
        out_shape=[
            jax.ShapeDtypeStruct((n, NUM_EXPERTS), jnp.float32),
            jax.ShapeDtypeStruct((NUM_EXPERTS, n), jnp.float32),
        ],
    )(x2d, wt)
    w1, w2, i1, i2 = _sc_topk(scores_t)
    return scores, jnp.stack([w1, w2], axis=1), jnp.stack([i1, i2], axis=1)


def kernel(x, W):
    x2d = x.reshape(-1, x.shape[-1])
    scores, weights, indices = _router(x2d, W.T)
    return scores, weights, indices, jnp.float32(0.0)
